# Initial kernel scaffold; baseline (speedup 1.0000x reference)
import jax
import jax.numpy as jnp
from jax.experimental import pallas as pl

N = 10000
D_Z = 256


def _zero_body(o_ref):
    o_ref[...] = jnp.zeros_like(o_ref)


def kernel(x, edge_index_atac, edge_index_rna, W1_a, b1_a, W1_r, b1_r, g1, be1, Wl0_a, Wr0_a, att0_a, bo0_a, Wl0_r, Wr0_r, att0_r, bo0_r, g2, be2, Wl1_a, Wr1_a, att1_a, bo1_a, Wl1_r, Wr1_r, att1_r, bo1_r, g3, be3, Wz_a, bz_a, Wz_r, bz_r):
    return pl.pallas_call(
        _zero_body,
        out_shape=jax.ShapeDtypeStruct((N, D_Z), jnp.float32),
    )()


# speedup vs baseline: 2.9163x; 2.9163x over previous
import functools

import jax
import jax.numpy as jnp
from jax import lax
from jax.experimental import pallas as pl
from jax.experimental.pallas import tpu as pltpu
from jax.experimental.pallas import tpu_sc as plsc

N = 10000
NPAD = 10240
D_IN = 512
D_H = 512
D_Z = 256
H = 4
HD = H * D_H
HP = 128
E = 80000
E_SL = E + N
E_PAD = 90112
NW = 32
L = 16
NODE_PER_W = NPAD // NW
FB = 1024

_MESH = plsc.VectorSubcoreMesh(core_axis_name="c", subcore_axis_name="s")
_SC_PARAMS = pltpu.CompilerParams(needs_layout_passes=False)


def _iota16():
    return lax.iota(jnp.int32, L)


@functools.partial(jax.jit, static_argnames=("d", "bg"))
def sc_gather(table, idx, *, d, bg):
    b_per_w = E_PAD // NW
    nbatch = b_per_w // bg

    def body(table_ref, idx_ref, out_ref, idx_v, rows_v, sem):
        wid = lax.axis_index("s") * 2 + lax.axis_index("c")
        base = wid * b_per_w

        def step(i, _):
            off = base + i * bg
            pltpu.sync_copy(idx_ref.at[pl.ds(off, bg)], idx_v)
            pltpu.async_copy(table_ref.at[idx_v], rows_v, sem).wait()
            pltpu.sync_copy(rows_v, out_ref.at[pl.ds(off, bg), :])
            return 0

        lax.fori_loop(0, nbatch, step, 0)

    return pl.kernel(
        body,
        out_type=jax.ShapeDtypeStruct((E_PAD, d), jnp.float32),
        mesh=_MESH,
        scratch_types=[
            pltpu.VMEM((bg,), jnp.int32),
            pltpu.VMEM((bg, d), jnp.float32),
            pltpu.SemaphoreType.DMA,
        ],
        compiler_params=_SC_PARAMS,
    )(table, idx)


@functools.partial(jax.jit, static_argnames=("d", "passes", "count_mode"))
def sc_gsa(row, col, table, *, d, passes, count_mode=False):
    nr = NODE_PER_W // passes
    nchunk = d // L
    nblocks = E_PAD // FB
    acc_words = (nr + L) * d

    def body(row_ref, col_ref, table_ref, out_ref, col_v, row_v, fcol, frow,
             acc, g_v, sem):
        wid = lax.axis_index("s") * 2 + lax.axis_index("c")
        zeros16 = jnp.zeros((L,), jnp.float32)
        ones16 = jnp.ones((L,), jnp.float32)

        for half in range(passes):
            base = wid * NODE_PER_W + half * nr

            def zstep(s, _):
                plsc.store_scatter(acc, [s * L + _iota16()], zeros16)
                return 0

            lax.fori_loop(0, acc_words // L, zstep, 0)

            def fblock(b, _):
                pltpu.sync_copy(col_ref.at[pl.ds(b * FB, FB)], col_v)
                if not count_mode:
                    pltpu.sync_copy(row_ref.at[pl.ds(b * FB, FB)], row_v)

                def fstep(j, cnt):
                    c16 = plsc.load_gather(col_v, [j * L + _iota16()])
                    loc = c16 - base
                    m = (loc >= 0) & (loc < nr)
                    ranks = plsc.cumsum(jnp.where(m, 1, 0))
                    idx = cnt + ranks - 1
                    plsc.store_scatter(fcol, [idx], loc, mask=m)
                    if not count_mode:
                        r16 = plsc.load_gather(row_v, [j * L + _iota16()])
                        plsc.store_scatter(frow, [idx], r16, mask=m)
                    return cnt + jnp.max(plsc.all_reduce_population_count(m))

                cnt = lax.fori_loop(0, FB // L, fstep, 0)
                plsc.store_scatter(fcol, [cnt + _iota16()],
                                   jnp.full((L,), nr, jnp.int32))
                if not count_mode:
                    plsc.store_scatter(frow, [cnt + _iota16()],
                                       jnp.zeros((L,), jnp.int32))

                def gbody(i, _):
                    if not count_mode:
                        pltpu.async_copy(
                            table_ref.at[frow.at[pl.ds(i * L, L)]], g_v, sem
                        ).wait()

                    def ebody(j, _):
                        ljv = plsc.load_gather(fcol, [i * L + j + _iota16() * 0])
                        if count_mode:
                            plsc.addupdate_scatter(
                                acc, [ljv * d + _iota16()], ones16)
                        else:
                            for c in range(nchunk):
                                gval = plsc.load_gather(
                                    g_v, [j + _iota16() * 0, c * L + _iota16()])
                                plsc.addupdate_scatter(
                                    acc, [ljv * d + c * L + _iota16()], gval)
                        return 0

                    lax.fori_loop(0, L, ebody, 0)
                    return 0

                lax.fori_loop(0, (cnt + L - 1) // L, gbody, 0)
                return 0

            lax.fori_loop(0, nblocks, fblock, 0)
            pltpu.sync_copy(acc.at[pl.ds(0, nr * d)],
                            out_ref.at[pl.ds(base * d, nr * d)])

    return pl.kernel(
        body,
        out_type=jax.ShapeDtypeStruct((NPAD * d,), jnp.float32),
        mesh=_MESH,
        scratch_types=[
            pltpu.VMEM((FB,), jnp.int32),
            pltpu.VMEM((FB,), jnp.int32),
            pltpu.VMEM((FB + L,), jnp.int32),
            pltpu.VMEM((FB + L,), jnp.int32),
            pltpu.VMEM((acc_words,), jnp.float32),
            pltpu.VMEM((L, d), jnp.float32),
            pltpu.SemaphoreType.DMA,
        ],
        compiler_params=_SC_PARAMS,
    )(row, col, table)


BN = 1024
BE = 1024


def _mm_body(x_ref, w_ref, o_ref):
    o_ref[...] = jnp.dot(x_ref[...], w_ref[...],
                         preferred_element_type=jnp.float32)


def _mm_dis_body(x_ref, w_ref, deg_ref, o_ref):
    y = jnp.dot(x_ref[...], w_ref[...], preferred_element_type=jnp.float32)
    deg = deg_ref[:, 0:1]
    dis = jnp.where(deg > 0, lax.rsqrt(jnp.maximum(deg, 1e-30)), 0.0)
    o_ref[...] = y * dis


@jax.jit
def tc_mm(x, w):
    n, k = x.shape
    dout = w.shape[1]
    bd = min(dout, 512)
    return pl.pallas_call(
        _mm_body,
        grid=(n // BN, dout // bd),
        in_specs=[
            pl.BlockSpec((BN, k), lambda i, j: (i, 0)),
            pl.BlockSpec((k, bd), lambda i, j: (0, j)),
        ],
        out_specs=pl.BlockSpec((BN, bd), lambda i, j: (i, j)),
        out_shape=jax.ShapeDtypeStruct((n, dout), jnp.float32),
    )(x, w)


@jax.jit
def tc_mm_dis(x, w, deg):
    n, k = x.shape
    dout = w.shape[1]
    bd = min(dout, 512)
    return pl.pallas_call(
        _mm_dis_body,
        grid=(n // BN, dout // bd),
        in_specs=[
            pl.BlockSpec((BN, k), lambda i, j: (i, 0)),
            pl.BlockSpec((k, bd), lambda i, j: (0, j)),
            pl.BlockSpec((BN, HP), lambda i, j: (i, 0)),
        ],
        out_specs=pl.BlockSpec((BN, bd), lambda i, j: (i, j)),
        out_shape=jax.ShapeDtypeStruct((n, dout), jnp.float32),
    )(x, w, deg)


def _alpha_body(xl_ref, xr_ref, att_ref, o_ref):
    s = xl_ref[...] + xr_ref[...]
    s = jnp.maximum(s, 0.2 * s)
    a = jnp.dot(s, att_ref[...], preferred_element_type=jnp.float32)
    o_ref[...] = jnp.exp(a)


@jax.jit
def tc_alpha(xg_l, xg_r, attp):
    return pl.pallas_call(
        _alpha_body,
        grid=(E_PAD // BE,),
        in_specs=[
            pl.BlockSpec((BE, HD), lambda i: (i, 0)),
            pl.BlockSpec((BE, HD), lambda i: (i, 0)),
            pl.BlockSpec((HD, HP), lambda i: (0, 0)),
        ],
        out_specs=pl.BlockSpec((BE, HP), lambda i: (i, 0)),
        out_shape=jax.ShapeDtypeStruct((E_PAD, HP), jnp.float32),
    )(xg_l, xg_r, attp)


def _vcomb_body(xl_ref, al_ref, asg_ref, o_ref):
    w = al_ref[...] / (asg_ref[...] + 1e-16) * (1.0 / H)
    acc = xl_ref[:, 0:D_H] * w[:, 0:1]
    for h in range(1, H):
        acc = acc + xl_ref[:, h * D_H:(h + 1) * D_H] * w[:, h:h + 1]
    o_ref[...] = acc


@jax.jit
def tc_vcomb(xg_l, al, asg):
    return pl.pallas_call(
        _vcomb_body,
        grid=(E_PAD // BE,),
        in_specs=[
            pl.BlockSpec((BE, HD), lambda i: (i, 0)),
            pl.BlockSpec((BE, HP), lambda i: (i, 0)),
            pl.BlockSpec((BE, HP), lambda i: (i, 0)),
        ],
        out_specs=pl.BlockSpec((BE, D_H), lambda i: (i, 0)),
        out_shape=jax.ShapeDtypeStruct((E_PAD, D_H), jnp.float32),
    )(xg_l, al, asg)


def _stats_body(h_ref, o_ref):
    i = pl.program_id(0)

    @pl.when(i == 0)
    def _():
        o_ref[...] = jnp.zeros_like(o_ref)

    rows = i * BN + lax.broadcasted_iota(jnp.int32, (BN, 1), 0)
    hv = jnp.where(rows < N, h_ref[...], 0.0)
    o_ref[0:1, :] += jnp.sum(hv, axis=0, keepdims=True)
    o_ref[1:2, :] += jnp.sum(hv * hv, axis=0, keepdims=True)


@jax.jit
def tc_stats(h):
    return pl.pallas_call(
        _stats_body,
        grid=(NPAD // BN,),
        in_specs=[pl.BlockSpec((BN, D_H), lambda i: (i, 0))],
        out_specs=pl.BlockSpec((8, D_H), lambda i: (0, 0)),
        out_shape=jax.ShapeDtypeStruct((8, D_H), jnp.float32),
    )(h)


def _normsilu_body(h_ref, st_ref, g_ref, be_ref, o_ref):
    m = st_ref[0:1, :] * (1.0 / N)
    var = st_ref[1:2, :] * (1.0 / N) - m * m
    rstd = lax.rsqrt(var + 1e-5)
    xn = (h_ref[...] - m) * rstd * g_ref[...] + be_ref[...]
    o_ref[...] = xn / (1.0 + jnp.exp(-xn))


@jax.jit
def tc_normsilu(h, st, g, be):
    return pl.pallas_call(
        _normsilu_body,
        grid=(NPAD // BN,),
        in_specs=[
            pl.BlockSpec((BN, D_H), lambda i: (i, 0)),
            pl.BlockSpec((8, D_H), lambda i: (0, 0)),
            pl.BlockSpec((1, D_H), lambda i: (0, 0)),
            pl.BlockSpec((1, D_H), lambda i: (0, 0)),
        ],
        out_specs=pl.BlockSpec((BN, D_H), lambda i: (i, 0)),
        out_shape=jax.ShapeDtypeStruct((NPAD, D_H), jnp.float32),
    )(h, st, g, be)


def _comb_gcn_body(aa_ref, dega_ref, ar_ref, degr_ref, b_ref, o_ref):
    da = dega_ref[:, 0:1]
    dra = jnp.where(da > 0, lax.rsqrt(jnp.maximum(da, 1e-30)), 0.0)
    dr = degr_ref[:, 0:1]
    drr = jnp.where(dr > 0, lax.rsqrt(jnp.maximum(dr, 1e-30)), 0.0)
    o_ref[...] = aa_ref[...] * dra + ar_ref[...] * drr + b_ref[...]


@jax.jit
def tc_comb_gcn(agg_a, deg_a, agg_r, deg_r, b):
    d = agg_a.shape[1]
    return pl.pallas_call(
        _comb_gcn_body,
        grid=(NPAD // BN,),
        in_specs=[
            pl.BlockSpec((BN, d), lambda i: (i, 0)),
            pl.BlockSpec((BN, HP), lambda i: (i, 0)),
            pl.BlockSpec((BN, d), lambda i: (i, 0)),
            pl.BlockSpec((BN, HP), lambda i: (i, 0)),
            pl.BlockSpec((1, d), lambda i: (0, 0)),
        ],
        out_specs=pl.BlockSpec((BN, d), lambda i: (i, 0)),
        out_shape=jax.ShapeDtypeStruct((NPAD, d), jnp.float32),
    )(agg_a, deg_a, agg_r, deg_r, b)


def _comb_add_body(a_ref, b_ref, c_ref, o_ref):
    o_ref[...] = a_ref[...] + b_ref[...] + c_ref[...]


@jax.jit
def tc_comb_add(a, b, bias):
    d = a.shape[1]
    return pl.pallas_call(
        _comb_add_body,
        grid=(NPAD // BN,),
        in_specs=[
            pl.BlockSpec((BN, d), lambda i: (i, 0)),
            pl.BlockSpec((BN, d), lambda i: (i, 0)),
            pl.BlockSpec((1, d), lambda i: (0, 0)),
        ],
        out_specs=pl.BlockSpec((BN, d), lambda i: (i, 0)),
        out_shape=jax.ShapeDtypeStruct((NPAD, d), jnp.float32),
    )(a, b, bias)


def _attp(att):
    return (att[:, :, None] * jnp.eye(H, HP, dtype=att.dtype)[:, None, :]
            ).reshape(HD, HP)


def _gat(hn, row, col, eiota, Wl, Wr, att):
    xl = tc_mm(hn, Wl)
    xr = tc_mm(hn, Wr)
    xg_l = sc_gather(xl, row, d=HD, bg=32)
    xg_r = sc_gather(xr, col, d=HD, bg=32)
    al = tc_alpha(xg_l, xg_r, _attp(att))
    asum = sc_gsa(eiota, col, al, d=HP, passes=1)
    asg = sc_gather(asum.reshape(NPAD, HP), col, d=HP, bg=128)
    v = tc_vcomb(xg_l, al, asg)
    return sc_gsa(eiota, col, v, d=D_H, passes=2).reshape(NPAD, D_H)


def kernel(x, edge_index_atac, edge_index_rna, W1_a, b1_a, W1_r, b1_r, g1, be1, Wl0_a, Wr0_a, att0_a, bo0_a, Wl0_r, Wr0_r, att0_r, bo0_r, g2, be2, Wl1_a, Wr1_a, att1_a, bo1_a, Wl1_r, Wr1_r, att1_r, bo1_r, g3, be3, Wz_a, bz_a, Wz_r, bz_r):
    i32 = jnp.int32
    loop = jnp.arange(N, dtype=i32)
    npadE = E_PAD - E_SL
    eiota = jnp.arange(E_PAD, dtype=i32)

    def prep(ei):
        row = jnp.concatenate([ei[0], loop, jnp.zeros((npadE,), i32)])
        col = jnp.concatenate([ei[1], loop, jnp.full((npadE,), NPAD - 1, i32)])
        return row, col

    row_a, col_a = prep(edge_index_atac)
    row_r, col_r = prep(edge_index_rna)

    xp = jnp.pad(x, ((0, NPAD - N), (0, 0)))

    dummy = jnp.zeros((8, HP), jnp.float32)
    deg_a = sc_gsa(eiota, col_a, dummy, d=HP, passes=1,
                   count_mode=True).reshape(NPAD, HP)
    deg_r = sc_gsa(eiota, col_r, dummy, d=HP, passes=1,
                   count_mode=True).reshape(NPAD, HP)

    def gcn(src, W, row, col, deg):
        y = tc_mm_dis(src, W, deg)
        d = W.shape[1]
        return sc_gsa(row, col, y, d=d,
                      passes=2 if d > 256 else 1).reshape(NPAD, d)

    agg_a = gcn(xp, W1_a, row_a, col_a, deg_a)
    agg_r = gcn(xp, W1_r, row_r, col_r, deg_r)
    h = tc_comb_gcn(agg_a, deg_a, agg_r, deg_r, (b1_a + b1_r).reshape(1, -1))
    h = tc_normsilu(h, tc_stats(h), g1.reshape(1, -1), be1.reshape(1, -1))

    for (Wl_a, Wr_a, att_a, bo_a, Wl_r, Wr_r, att_r, bo_r, g, be) in (
        (Wl0_a, Wr0_a, att0_a, bo0_a, Wl0_r, Wr0_r, att0_r, bo0_r, g2, be2),
        (Wl1_a, Wr1_a, att1_a, bo1_a, Wl1_r, Wr1_r, att1_r, bo1_r, g3, be3),
    ):
        ga = _gat(h, row_a, col_a, eiota, Wl_a, Wr_a, att_a)
        gr = _gat(h, row_r, col_r, eiota, Wl_r, Wr_r, att_r)
        h = tc_comb_add(ga, gr, (bo_a + bo_r).reshape(1, -1))
        h = tc_normsilu(h, tc_stats(h), g.reshape(1, -1), be.reshape(1, -1))

    za = gcn(h, Wz_a, row_a, col_a, deg_a)
    zr = gcn(h, Wz_r, row_r, col_r, deg_r)
    z = tc_comb_gcn(za, deg_a, zr, deg_r, (bz_a + bz_r).reshape(1, -1))
    return z[:N]


# speedup vs baseline: 3.5476x; 1.2165x over previous
import functools

import jax
import jax.numpy as jnp
from jax import lax
from jax.experimental import pallas as pl
from jax.experimental.pallas import tpu as pltpu
from jax.experimental.pallas import tpu_sc as plsc

N = 10000
NPAD = 10240
D_IN = 512
D_H = 512
D_Z = 256
H = 4
HD = H * D_H
HP = 128
E = 80000
E_SL = E + N
E_PAD = 90112
NW = 32
L = 16
NODE_PER_W = NPAD // NW
FB = 1024

_MESH = plsc.VectorSubcoreMesh(core_axis_name="c", subcore_axis_name="s")
_SC_PARAMS = pltpu.CompilerParams(needs_layout_passes=False)


def _iota16():
    return lax.iota(jnp.int32, L)


@functools.partial(jax.jit, static_argnames=("d", "bg"))
def sc_gather(table, idx, *, d, bg):
    b_per_w = E_PAD // NW
    nbatch = b_per_w // bg

    def body(table_ref, idx_ref, out_ref, idx_v, rows_v, sem):
        wid = lax.axis_index("s") * 2 + lax.axis_index("c")
        base = wid * b_per_w

        def step(i, _):
            off = base + i * bg
            pltpu.sync_copy(idx_ref.at[pl.ds(off, bg)], idx_v)
            pltpu.async_copy(table_ref.at[idx_v], rows_v, sem).wait()
            pltpu.sync_copy(rows_v, out_ref.at[pl.ds(off, bg), :])
            return 0

        lax.fori_loop(0, nbatch, step, 0)

    return pl.kernel(
        body,
        out_type=jax.ShapeDtypeStruct((E_PAD, d), jnp.float32),
        mesh=_MESH,
        scratch_types=[
            pltpu.VMEM((bg,), jnp.int32),
            pltpu.VMEM((bg, d), jnp.float32),
            pltpu.SemaphoreType.DMA,
        ],
        compiler_params=_SC_PARAMS,
    )(table, idx)


@functools.partial(jax.jit, static_argnames=("d", "passes", "count_mode"))
def sc_gsa(row, col, table, *, d, passes, count_mode=False):
    QD = d // HP
    bw = HP // QD
    nr = NPAD // 2 // passes
    assert nr * QD == NPAD // 2 * QD // passes and nr * QD == 5120
    AR = nr * QD + 128
    zr = AR // 16
    fz = nr * QD // 16
    e_per_t = E_PAD // 16
    fb = 512
    nblocks = e_per_t // fb

    def body(row_ref, col_ref, t4_ref, aux_ref, out_ref, col_v, row_v,
             fcol, frow, fr4, idx4, g_v, accS, sem):
        cid = lax.axis_index("c")
        sid = lax.axis_index("s")
        tbase = sid * e_per_t

        if count_mode:
            pltpu.sync_copy(aux_ref.at[pl.ds(zr, HP), :], g_v)

        for p in range(passes):
            base = cid * (NPAD // 2) + p * nr

            pltpu.sync_copy(aux_ref.at[pl.ds(0, zr), :],
                            accS.at[pl.ds(sid * zr, zr), :])
            plsc.subcore_barrier()

            def fblock(blk, _):
                off = tbase + blk * fb
                pltpu.sync_copy(col_ref.at[pl.ds(off, fb)], col_v)
                if not count_mode:
                    pltpu.sync_copy(row_ref.at[pl.ds(off, fb)], row_v)

                def fstep(j, cnt):
                    c16 = plsc.load_gather(col_v, [j * L + _iota16()])
                    loc = c16 - base
                    m = (loc >= 0) & (loc < nr)
                    ranks = plsc.cumsum(jnp.where(m, 1, 0))
                    idx = cnt + ranks - 1
                    plsc.store_scatter(fcol, [idx], loc, mask=m)
                    if not count_mode:
                        r16 = plsc.load_gather(row_v, [j * L + _iota16()])
                        plsc.store_scatter(frow, [idx], r16, mask=m)
                    return cnt + jnp.max(plsc.all_reduce_population_count(m))

                cnt = lax.fori_loop(0, fb // L, fstep, 0)
                for k in range(bw // L):
                    pidx = cnt + k * L + _iota16()
                    plsc.store_scatter(fcol, [pidx],
                                       jnp.full((L,), nr, jnp.int32))
                    if not count_mode:
                        plsc.store_scatter(frow, [pidx],
                                           jnp.zeros((L,), jnp.int32))

                def gbody(i, _):
                    for k in range(bw // L):
                        locv = plsc.load_gather(
                            fcol, [i * bw + k * L + _iota16()])
                        if not count_mode:
                            rowv = plsc.load_gather(
                                frow, [i * bw + k * L + _iota16()])
                        for q in range(QD):
                            pos = (k * L + _iota16()) * QD + q
                            plsc.store_scatter(idx4, [pos], locv * QD + q)
                            if not count_mode:
                                plsc.store_scatter(fr4, [pos], rowv * QD + q)
                    if not count_mode:
                        pltpu.async_copy(t4_ref.at[fr4], g_v, sem).wait()
                    pltpu.sync_copy(g_v, accS.at[idx4], add=True)
                    return 0

                lax.fori_loop(0, (cnt + bw - 1) // bw, gbody, 0)
                return 0

            lax.fori_loop(0, nblocks, fblock, 0)
            plsc.subcore_barrier()
            pltpu.sync_copy(accS.at[pl.ds(sid * fz, fz), :],
                            out_ref.at[pl.ds(base * QD + sid * fz, fz), :])
            if passes > 1:
                plsc.subcore_barrier()

    aux = jnp.concatenate([jnp.zeros((zr, HP), jnp.float32),
                           jnp.ones((HP, HP), jnp.float32)])

    out4 = pl.kernel(
        body,
        out_type=jax.ShapeDtypeStruct((NPAD * QD, HP), jnp.float32),
        mesh=_MESH,
        scratch_types=[
            pltpu.VMEM((fb,), jnp.int32),
            pltpu.VMEM((fb,), jnp.int32),
            pltpu.VMEM((fb + bw,), jnp.int32),
            pltpu.VMEM((fb + bw,), jnp.int32),
            pltpu.VMEM((HP,), jnp.int32),
            pltpu.VMEM((HP,), jnp.int32),
            pltpu.VMEM((HP, HP), jnp.float32),
            pltpu.VMEM_SHARED((AR, HP), jnp.float32),
            pltpu.SemaphoreType.DMA,
        ],
        compiler_params=_SC_PARAMS,
    )(row, col, table.reshape(-1, HP), aux)
    return out4.reshape(NPAD, d)


BN = 1024
BE = 1024


def _mm_body(x_ref, w_ref, o_ref):
    o_ref[...] = jnp.dot(x_ref[...], w_ref[...],
                         preferred_element_type=jnp.float32)


def _mm_dis_body(x_ref, w_ref, deg_ref, o_ref):
    y = jnp.dot(x_ref[...], w_ref[...], preferred_element_type=jnp.float32)
    deg = deg_ref[:, 0:1]
    dis = jnp.where(deg > 0, lax.rsqrt(jnp.maximum(deg, 1e-30)), 0.0)
    o_ref[...] = y * dis


@jax.jit
def tc_mm(x, w):
    n, k = x.shape
    dout = w.shape[1]
    bd = min(dout, 512)
    return pl.pallas_call(
        _mm_body,
        grid=(n // BN, dout // bd),
        in_specs=[
            pl.BlockSpec((BN, k), lambda i, j: (i, 0)),
            pl.BlockSpec((k, bd), lambda i, j: (0, j)),
        ],
        out_specs=pl.BlockSpec((BN, bd), lambda i, j: (i, j)),
        out_shape=jax.ShapeDtypeStruct((n, dout), jnp.float32),
    )(x, w)


@jax.jit
def tc_mm_dis(x, w, deg):
    n, k = x.shape
    dout = w.shape[1]
    bd = min(dout, 512)
    return pl.pallas_call(
        _mm_dis_body,
        grid=(n // BN, dout // bd),
        in_specs=[
            pl.BlockSpec((BN, k), lambda i, j: (i, 0)),
            pl.BlockSpec((k, bd), lambda i, j: (0, j)),
            pl.BlockSpec((BN, HP), lambda i, j: (i, 0)),
        ],
        out_specs=pl.BlockSpec((BN, bd), lambda i, j: (i, j)),
        out_shape=jax.ShapeDtypeStruct((n, dout), jnp.float32),
    )(x, w, deg)


def _alpha_body(xl_ref, xr_ref, att_ref, o_ref):
    s = xl_ref[...] + xr_ref[...]
    s = jnp.maximum(s, 0.2 * s)
    a = jnp.dot(s, att_ref[...], preferred_element_type=jnp.float32)
    o_ref[...] = jnp.exp(a)


@jax.jit
def tc_alpha(xg_l, xg_r, attp):
    return pl.pallas_call(
        _alpha_body,
        grid=(E_PAD // BE,),
        in_specs=[
            pl.BlockSpec((BE, HD), lambda i: (i, 0)),
            pl.BlockSpec((BE, HD), lambda i: (i, 0)),
            pl.BlockSpec((HD, HP), lambda i: (0, 0)),
        ],
        out_specs=pl.BlockSpec((BE, HP), lambda i: (i, 0)),
        out_shape=jax.ShapeDtypeStruct((E_PAD, HP), jnp.float32),
    )(xg_l, xg_r, attp)


def _vcomb_body(xl_ref, al_ref, asg_ref, o_ref):
    w = al_ref[...] / (asg_ref[...] + 1e-16) * (1.0 / H)
    acc = xl_ref[:, 0:D_H] * w[:, 0:1]
    for h in range(1, H):
        acc = acc + xl_ref[:, h * D_H:(h + 1) * D_H] * w[:, h:h + 1]
    o_ref[...] = acc


@jax.jit
def tc_vcomb(xg_l, al, asg):
    return pl.pallas_call(
        _vcomb_body,
        grid=(E_PAD // BE,),
        in_specs=[
            pl.BlockSpec((BE, HD), lambda i: (i, 0)),
            pl.BlockSpec((BE, HP), lambda i: (i, 0)),
            pl.BlockSpec((BE, HP), lambda i: (i, 0)),
        ],
        out_specs=pl.BlockSpec((BE, D_H), lambda i: (i, 0)),
        out_shape=jax.ShapeDtypeStruct((E_PAD, D_H), jnp.float32),
    )(xg_l, al, asg)


def _stats_body(h_ref, o_ref):
    i = pl.program_id(0)

    @pl.when(i == 0)
    def _():
        o_ref[...] = jnp.zeros_like(o_ref)

    rows = i * BN + lax.broadcasted_iota(jnp.int32, (BN, 1), 0)
    hv = jnp.where(rows < N, h_ref[...], 0.0)
    o_ref[0:1, :] += jnp.sum(hv, axis=0, keepdims=True)
    o_ref[1:2, :] += jnp.sum(hv * hv, axis=0, keepdims=True)


@jax.jit
def tc_stats(h):
    return pl.pallas_call(
        _stats_body,
        grid=(NPAD // BN,),
        in_specs=[pl.BlockSpec((BN, D_H), lambda i: (i, 0))],
        out_specs=pl.BlockSpec((8, D_H), lambda i: (0, 0)),
        out_shape=jax.ShapeDtypeStruct((8, D_H), jnp.float32),
    )(h)


def _normsilu_body(h_ref, st_ref, g_ref, be_ref, o_ref):
    m = st_ref[0:1, :] * (1.0 / N)
    var = st_ref[1:2, :] * (1.0 / N) - m * m
    rstd = lax.rsqrt(var + 1e-5)
    xn = (h_ref[...] - m) * rstd * g_ref[...] + be_ref[...]
    o_ref[...] = xn / (1.0 + jnp.exp(-xn))


@jax.jit
def tc_normsilu(h, st, g, be):
    return pl.pallas_call(
        _normsilu_body,
        grid=(NPAD // BN,),
        in_specs=[
            pl.BlockSpec((BN, D_H), lambda i: (i, 0)),
            pl.BlockSpec((8, D_H), lambda i: (0, 0)),
            pl.BlockSpec((1, D_H), lambda i: (0, 0)),
            pl.BlockSpec((1, D_H), lambda i: (0, 0)),
        ],
        out_specs=pl.BlockSpec((BN, D_H), lambda i: (i, 0)),
        out_shape=jax.ShapeDtypeStruct((NPAD, D_H), jnp.float32),
    )(h, st, g, be)


def _comb_gcn_body(aa_ref, dega_ref, ar_ref, degr_ref, b_ref, o_ref):
    da = dega_ref[:, 0:1]
    dra = jnp.where(da > 0, lax.rsqrt(jnp.maximum(da, 1e-30)), 0.0)
    dr = degr_ref[:, 0:1]
    drr = jnp.where(dr > 0, lax.rsqrt(jnp.maximum(dr, 1e-30)), 0.0)
    o_ref[...] = aa_ref[...] * dra + ar_ref[...] * drr + b_ref[...]


@jax.jit
def tc_comb_gcn(agg_a, deg_a, agg_r, deg_r, b):
    d = agg_a.shape[1]
    return pl.pallas_call(
        _comb_gcn_body,
        grid=(NPAD // BN,),
        in_specs=[
            pl.BlockSpec((BN, d), lambda i: (i, 0)),
            pl.BlockSpec((BN, HP), lambda i: (i, 0)),
            pl.BlockSpec((BN, d), lambda i: (i, 0)),
            pl.BlockSpec((BN, HP), lambda i: (i, 0)),
            pl.BlockSpec((1, d), lambda i: (0, 0)),
        ],
        out_specs=pl.BlockSpec((BN, d), lambda i: (i, 0)),
        out_shape=jax.ShapeDtypeStruct((NPAD, d), jnp.float32),
    )(agg_a, deg_a, agg_r, deg_r, b)


def _comb_add_body(a_ref, b_ref, c_ref, o_ref):
    o_ref[...] = a_ref[...] + b_ref[...] + c_ref[...]


@jax.jit
def tc_comb_add(a, b, bias):
    d = a.shape[1]
    return pl.pallas_call(
        _comb_add_body,
        grid=(NPAD // BN,),
        in_specs=[
            pl.BlockSpec((BN, d), lambda i: (i, 0)),
            pl.BlockSpec((BN, d), lambda i: (i, 0)),
            pl.BlockSpec((1, d), lambda i: (0, 0)),
        ],
        out_specs=pl.BlockSpec((BN, d), lambda i: (i, 0)),
        out_shape=jax.ShapeDtypeStruct((NPAD, d), jnp.float32),
    )(a, b, bias)


def _attp(att):
    return (att[:, :, None] * jnp.eye(H, HP, dtype=att.dtype)[:, None, :]
            ).reshape(HD, HP)


def _gat(hn, row, col, eiota, Wl, Wr, att):
    xl = tc_mm(hn, Wl)
    xr = tc_mm(hn, Wr)
    xg_l = sc_gather(xl, row, d=HD, bg=32)
    xg_r = sc_gather(xr, col, d=HD, bg=32)
    al = tc_alpha(xg_l, xg_r, _attp(att))
    asum = sc_gsa(eiota, col, al, d=HP, passes=1)
    asg = sc_gather(asum, col, d=HP, bg=128)
    v = tc_vcomb(xg_l, al, asg)
    return sc_gsa(eiota, col, v, d=D_H, passes=4)


def kernel(x, edge_index_atac, edge_index_rna, W1_a, b1_a, W1_r, b1_r, g1, be1, Wl0_a, Wr0_a, att0_a, bo0_a, Wl0_r, Wr0_r, att0_r, bo0_r, g2, be2, Wl1_a, Wr1_a, att1_a, bo1_a, Wl1_r, Wr1_r, att1_r, bo1_r, g3, be3, Wz_a, bz_a, Wz_r, bz_r):
    i32 = jnp.int32
    loop = jnp.arange(N, dtype=i32)
    npadE = E_PAD - E_SL
    eiota = jnp.arange(E_PAD, dtype=i32)

    def prep(ei):
        row = jnp.concatenate([ei[0], loop, jnp.zeros((npadE,), i32)])
        col = jnp.concatenate([ei[1], loop, jnp.full((npadE,), NPAD - 1, i32)])
        return row, col

    row_a, col_a = prep(edge_index_atac)
    row_r, col_r = prep(edge_index_rna)

    xp = jnp.pad(x, ((0, NPAD - N), (0, 0)))

    dummy = jnp.zeros((8, HP), jnp.float32)
    deg_a = sc_gsa(eiota, col_a, dummy, d=HP, passes=1, count_mode=True)
    deg_r = sc_gsa(eiota, col_r, dummy, d=HP, passes=1, count_mode=True)

    def gcn(src, W, row, col, deg):
        y = tc_mm_dis(src, W, deg)
        d = W.shape[1]
        return sc_gsa(row, col, y, d=d, passes=4 if d > 256 else 2)

    agg_a = gcn(xp, W1_a, row_a, col_a, deg_a)
    agg_r = gcn(xp, W1_r, row_r, col_r, deg_r)
    h = tc_comb_gcn(agg_a, deg_a, agg_r, deg_r, (b1_a + b1_r).reshape(1, -1))
    h = tc_normsilu(h, tc_stats(h), g1.reshape(1, -1), be1.reshape(1, -1))

    for (Wl_a, Wr_a, att_a, bo_a, Wl_r, Wr_r, att_r, bo_r, g, be) in (
        (Wl0_a, Wr0_a, att0_a, bo0_a, Wl0_r, Wr0_r, att0_r, bo0_r, g2, be2),
        (Wl1_a, Wr1_a, att1_a, bo1_a, Wl1_r, Wr1_r, att1_r, bo1_r, g3, be3),
    ):
        ga = _gat(h, row_a, col_a, eiota, Wl_a, Wr_a, att_a)
        gr = _gat(h, row_r, col_r, eiota, Wl_r, Wr_r, att_r)
        h = tc_comb_add(ga, gr, (bo_a + bo_r).reshape(1, -1))
        h = tc_normsilu(h, tc_stats(h), g.reshape(1, -1), be.reshape(1, -1))

    za = gcn(h, Wz_a, row_a, col_a, deg_a)
    zr = gcn(h, Wz_r, row_r, col_r, deg_r)
    z = tc_comb_gcn(za, deg_a, zr, deg_r, (bz_a + bz_r).reshape(1, -1))
    return z[:N]


# speedup vs baseline: 3.5876x; 1.0113x over previous
import functools

import jax
import jax.numpy as jnp
from jax import lax
from jax.experimental import pallas as pl
from jax.experimental.pallas import tpu as pltpu
from jax.experimental.pallas import tpu_sc as plsc

N = 10000
NPAD = 10240
D_IN = 512
D_H = 512
D_Z = 256
H = 4
HD = H * D_H
HP = 128
E = 80000
E_SL = E + N
E_PAD = 90112
NW = 32
L = 16
NODE_PER_W = NPAD // NW
FB = 1024

_MESH = plsc.VectorSubcoreMesh(core_axis_name="c", subcore_axis_name="s")
_SC_PARAMS = pltpu.CompilerParams(needs_layout_passes=False)


def _iota16():
    return lax.iota(jnp.int32, L)


@functools.partial(jax.jit, static_argnames=("d", "bg"))
def sc_gather(table, idx, *, d, bg):
    b_per_w = E_PAD // NW
    nbatch = b_per_w // bg

    def body(table_ref, idx_ref, out_ref, idx_a, idx_b, rows_a, rows_b,
             sem_a, sem_b):
        wid = lax.axis_index("s") * 2 + lax.axis_index("c")
        base = wid * b_per_w

        def step(i, _):
            o1 = base + 2 * i * bg
            o2 = o1 + bg
            pltpu.sync_copy(idx_ref.at[pl.ds(o1, bg)], idx_a)
            cp_a = pltpu.async_copy(table_ref.at[idx_a], rows_a, sem_a)
            pltpu.sync_copy(idx_ref.at[pl.ds(o2, bg)], idx_b)
            cp_b = pltpu.async_copy(table_ref.at[idx_b], rows_b, sem_b)
            cp_a.wait()
            pltpu.sync_copy(rows_a, out_ref.at[pl.ds(o1, bg), :])
            cp_b.wait()
            pltpu.sync_copy(rows_b, out_ref.at[pl.ds(o2, bg), :])
            return 0

        lax.fori_loop(0, nbatch // 2, step, 0)

    return pl.kernel(
        body,
        out_type=jax.ShapeDtypeStruct((E_PAD, d), jnp.float32),
        mesh=_MESH,
        scratch_types=[
            pltpu.VMEM((bg,), jnp.int32),
            pltpu.VMEM((bg,), jnp.int32),
            pltpu.VMEM((bg, d), jnp.float32),
            pltpu.VMEM((bg, d), jnp.float32),
            pltpu.SemaphoreType.DMA,
            pltpu.SemaphoreType.DMA,
        ],
        compiler_params=_SC_PARAMS,
    )(table, idx)


@functools.partial(jax.jit, static_argnames=("d", "passes", "count_mode"))
def sc_gsa(row, col, table, *, d, passes, count_mode=False):
    QD = d // HP
    bw = HP // QD
    nr = NPAD // 2 // passes
    assert nr * QD == NPAD // 2 * QD // passes and nr * QD == 5120
    AR = nr * QD + 128
    zr = AR // 16
    fz = nr * QD // 16
    e_per_t = E_PAD // 16
    fb = 512
    nblocks = e_per_t // fb

    def body(row_ref, col_ref, t4_ref, aux_ref, out_ref, col_v, row_v,
             fcol, frow, fr4, idx4, g_v, accS, sem):
        cid = lax.axis_index("c")
        sid = lax.axis_index("s")
        tbase = sid * e_per_t

        if count_mode:
            pltpu.sync_copy(aux_ref.at[pl.ds(zr, HP), :], g_v)

        for p in range(passes):
            base = cid * (NPAD // 2) + p * nr

            pltpu.sync_copy(aux_ref.at[pl.ds(0, zr), :],
                            accS.at[pl.ds(sid * zr, zr), :])
            plsc.subcore_barrier()

            def fblock(blk, _):
                off = tbase + blk * fb
                pltpu.sync_copy(col_ref.at[pl.ds(off, fb)], col_v)
                if not count_mode:
                    pltpu.sync_copy(row_ref.at[pl.ds(off, fb)], row_v)

                def fstep(j, cnt):
                    c16 = plsc.load_gather(col_v, [j * L + _iota16()])
                    loc = c16 - base
                    m = (loc >= 0) & (loc < nr)
                    ranks = plsc.cumsum(jnp.where(m, 1, 0))
                    idx = cnt + ranks - 1
                    plsc.store_scatter(fcol, [idx], loc, mask=m)
                    if not count_mode:
                        r16 = plsc.load_gather(row_v, [j * L + _iota16()])
                        plsc.store_scatter(frow, [idx], r16, mask=m)
                    return cnt + jnp.max(plsc.all_reduce_population_count(m))

                cnt = lax.fori_loop(0, fb // L, fstep, 0)
                for k in range(bw // L):
                    pidx = cnt + k * L + _iota16()
                    plsc.store_scatter(fcol, [pidx],
                                       jnp.full((L,), nr, jnp.int32))
                    if not count_mode:
                        plsc.store_scatter(frow, [pidx],
                                           jnp.zeros((L,), jnp.int32))

                def gbody(i, _):
                    for k in range(bw // L):
                        locv = plsc.load_gather(
                            fcol, [i * bw + k * L + _iota16()])
                        if not count_mode:
                            rowv = plsc.load_gather(
                                frow, [i * bw + k * L + _iota16()])
                        for q in range(QD):
                            pos = (k * L + _iota16()) * QD + q
                            plsc.store_scatter(idx4, [pos], locv * QD + q)
                            if not count_mode:
                                plsc.store_scatter(fr4, [pos], rowv * QD + q)
                    if not count_mode:
                        pltpu.async_copy(t4_ref.at[fr4], g_v, sem).wait()
                    pltpu.sync_copy(g_v, accS.at[idx4], add=True)
                    return 0

                lax.fori_loop(0, (cnt + bw - 1) // bw, gbody, 0)
                return 0

            lax.fori_loop(0, nblocks, fblock, 0)
            plsc.subcore_barrier()
            pltpu.sync_copy(accS.at[pl.ds(sid * fz, fz), :],
                            out_ref.at[pl.ds(base * QD + sid * fz, fz), :])
            if passes > 1:
                plsc.subcore_barrier()

    aux = jnp.concatenate([jnp.zeros((zr, HP), jnp.float32),
                           jnp.ones((HP, HP), jnp.float32)])

    out4 = pl.kernel(
        body,
        out_type=jax.ShapeDtypeStruct((NPAD * QD, HP), jnp.float32),
        mesh=_MESH,
        scratch_types=[
            pltpu.VMEM((fb,), jnp.int32),
            pltpu.VMEM((fb,), jnp.int32),
            pltpu.VMEM((fb + bw,), jnp.int32),
            pltpu.VMEM((fb + bw,), jnp.int32),
            pltpu.VMEM((HP,), jnp.int32),
            pltpu.VMEM((HP,), jnp.int32),
            pltpu.VMEM((HP, HP), jnp.float32),
            pltpu.VMEM_SHARED((AR, HP), jnp.float32),
            pltpu.SemaphoreType.DMA,
        ],
        compiler_params=_SC_PARAMS,
    )(row, col, table.reshape(-1, HP), aux)
    return out4.reshape(NPAD, d)


BN = 1024
BE = 1024


def _mm_body(x_ref, w_ref, o_ref):
    o_ref[...] = jnp.dot(x_ref[...], w_ref[...],
                         preferred_element_type=jnp.float32)


def _mm_dis_body(x_ref, w_ref, deg_ref, o_ref):
    y = jnp.dot(x_ref[...], w_ref[...], preferred_element_type=jnp.float32)
    deg = deg_ref[:, 0:1]
    dis = jnp.where(deg > 0, lax.rsqrt(jnp.maximum(deg, 1e-30)), 0.0)
    o_ref[...] = y * dis


@jax.jit
def tc_mm(x, w):
    n, k = x.shape
    dout = w.shape[1]
    bd = min(dout, 512)
    return pl.pallas_call(
        _mm_body,
        grid=(n // BN, dout // bd),
        in_specs=[
            pl.BlockSpec((BN, k), lambda i, j: (i, 0)),
            pl.BlockSpec((k, bd), lambda i, j: (0, j)),
        ],
        out_specs=pl.BlockSpec((BN, bd), lambda i, j: (i, j)),
        out_shape=jax.ShapeDtypeStruct((n, dout), jnp.float32),
    )(x, w)


@jax.jit
def tc_mm_dis(x, w, deg):
    n, k = x.shape
    dout = w.shape[1]
    bd = min(dout, 512)
    return pl.pallas_call(
        _mm_dis_body,
        grid=(n // BN, dout // bd),
        in_specs=[
            pl.BlockSpec((BN, k), lambda i, j: (i, 0)),
            pl.BlockSpec((k, bd), lambda i, j: (0, j)),
            pl.BlockSpec((BN, HP), lambda i, j: (i, 0)),
        ],
        out_specs=pl.BlockSpec((BN, bd), lambda i, j: (i, j)),
        out_shape=jax.ShapeDtypeStruct((n, dout), jnp.float32),
    )(x, w, deg)


def _alpha_body(xl_ref, xr_ref, att_ref, o_ref):
    s = xl_ref[...] + xr_ref[...]
    s = jnp.maximum(s, 0.2 * s)
    a = jnp.dot(s, att_ref[...], preferred_element_type=jnp.float32)
    o_ref[...] = jnp.exp(a)


@jax.jit
def tc_alpha(xg_l, xg_r, attp):
    return pl.pallas_call(
        _alpha_body,
        grid=(E_PAD // BE,),
        in_specs=[
            pl.BlockSpec((BE, HD), lambda i: (i, 0)),
            pl.BlockSpec((BE, HD), lambda i: (i, 0)),
            pl.BlockSpec((HD, HP), lambda i: (0, 0)),
        ],
        out_specs=pl.BlockSpec((BE, HP), lambda i: (i, 0)),
        out_shape=jax.ShapeDtypeStruct((E_PAD, HP), jnp.float32),
    )(xg_l, xg_r, attp)


def _vcomb_body(xl_ref, al_ref, asg_ref, o_ref):
    w = al_ref[...] / (asg_ref[...] + 1e-16) * (1.0 / H)
    acc = xl_ref[:, 0:D_H] * w[:, 0:1]
    for h in range(1, H):
        acc = acc + xl_ref[:, h * D_H:(h + 1) * D_H] * w[:, h:h + 1]
    o_ref[...] = acc


@jax.jit
def tc_vcomb(xg_l, al, asg):
    return pl.pallas_call(
        _vcomb_body,
        grid=(E_PAD // BE,),
        in_specs=[
            pl.BlockSpec((BE, HD), lambda i: (i, 0)),
            pl.BlockSpec((BE, HP), lambda i: (i, 0)),
            pl.BlockSpec((BE, HP), lambda i: (i, 0)),
        ],
        out_specs=pl.BlockSpec((BE, D_H), lambda i: (i, 0)),
        out_shape=jax.ShapeDtypeStruct((E_PAD, D_H), jnp.float32),
    )(xg_l, al, asg)


def _stats_body(h_ref, o_ref):
    i = pl.program_id(0)

    @pl.when(i == 0)
    def _():
        o_ref[...] = jnp.zeros_like(o_ref)

    rows = i * BN + lax.broadcasted_iota(jnp.int32, (BN, 1), 0)
    hv = jnp.where(rows < N, h_ref[...], 0.0)
    o_ref[0:1, :] += jnp.sum(hv, axis=0, keepdims=True)
    o_ref[1:2, :] += jnp.sum(hv * hv, axis=0, keepdims=True)


@jax.jit
def tc_stats(h):
    return pl.pallas_call(
        _stats_body,
        grid=(NPAD // BN,),
        in_specs=[pl.BlockSpec((BN, D_H), lambda i: (i, 0))],
        out_specs=pl.BlockSpec((8, D_H), lambda i: (0, 0)),
        out_shape=jax.ShapeDtypeStruct((8, D_H), jnp.float32),
    )(h)


def _normsilu_body(h_ref, st_ref, g_ref, be_ref, o_ref):
    m = st_ref[0:1, :] * (1.0 / N)
    var = st_ref[1:2, :] * (1.0 / N) - m * m
    rstd = lax.rsqrt(var + 1e-5)
    xn = (h_ref[...] - m) * rstd * g_ref[...] + be_ref[...]
    o_ref[...] = xn / (1.0 + jnp.exp(-xn))


@jax.jit
def tc_normsilu(h, st, g, be):
    return pl.pallas_call(
        _normsilu_body,
        grid=(NPAD // BN,),
        in_specs=[
            pl.BlockSpec((BN, D_H), lambda i: (i, 0)),
            pl.BlockSpec((8, D_H), lambda i: (0, 0)),
            pl.BlockSpec((1, D_H), lambda i: (0, 0)),
            pl.BlockSpec((1, D_H), lambda i: (0, 0)),
        ],
        out_specs=pl.BlockSpec((BN, D_H), lambda i: (i, 0)),
        out_shape=jax.ShapeDtypeStruct((NPAD, D_H), jnp.float32),
    )(h, st, g, be)


def _comb_gcn_body(aa_ref, dega_ref, ar_ref, degr_ref, b_ref, o_ref):
    da = dega_ref[:, 0:1]
    dra = jnp.where(da > 0, lax.rsqrt(jnp.maximum(da, 1e-30)), 0.0)
    dr = degr_ref[:, 0:1]
    drr = jnp.where(dr > 0, lax.rsqrt(jnp.maximum(dr, 1e-30)), 0.0)
    o_ref[...] = aa_ref[...] * dra + ar_ref[...] * drr + b_ref[...]


@jax.jit
def tc_comb_gcn(agg_a, deg_a, agg_r, deg_r, b):
    d = agg_a.shape[1]
    return pl.pallas_call(
        _comb_gcn_body,
        grid=(NPAD // BN,),
        in_specs=[
            pl.BlockSpec((BN, d), lambda i: (i, 0)),
            pl.BlockSpec((BN, HP), lambda i: (i, 0)),
            pl.BlockSpec((BN, d), lambda i: (i, 0)),
            pl.BlockSpec((BN, HP), lambda i: (i, 0)),
            pl.BlockSpec((1, d), lambda i: (0, 0)),
        ],
        out_specs=pl.BlockSpec((BN, d), lambda i: (i, 0)),
        out_shape=jax.ShapeDtypeStruct((NPAD, d), jnp.float32),
    )(agg_a, deg_a, agg_r, deg_r, b)


def _comb_add_body(a_ref, b_ref, c_ref, o_ref):
    o_ref[...] = a_ref[...] + b_ref[...] + c_ref[...]


@jax.jit
def tc_comb_add(a, b, bias):
    d = a.shape[1]
    return pl.pallas_call(
        _comb_add_body,
        grid=(NPAD // BN,),
        in_specs=[
            pl.BlockSpec((BN, d), lambda i: (i, 0)),
            pl.BlockSpec((BN, d), lambda i: (i, 0)),
            pl.BlockSpec((1, d), lambda i: (0, 0)),
        ],
        out_specs=pl.BlockSpec((BN, d), lambda i: (i, 0)),
        out_shape=jax.ShapeDtypeStruct((NPAD, d), jnp.float32),
    )(a, b, bias)


def _attp(att):
    return (att[:, :, None] * jnp.eye(H, HP, dtype=att.dtype)[:, None, :]
            ).reshape(HD, HP)


def _gat(hn, row, col, eiota, Wl, Wr, att):
    xl = tc_mm(hn, Wl)
    xr = tc_mm(hn, Wr)
    xg_l = sc_gather(xl, row, d=HD, bg=16)
    xg_r = sc_gather(xr, col, d=HD, bg=16)
    al = tc_alpha(xg_l, xg_r, _attp(att))
    asum = sc_gsa(eiota, col, al, d=HP, passes=1)
    asg = sc_gather(asum, col, d=HP, bg=128)
    v = tc_vcomb(xg_l, al, asg)
    return sc_gsa(eiota, col, v, d=D_H, passes=4)


def kernel(x, edge_index_atac, edge_index_rna, W1_a, b1_a, W1_r, b1_r, g1, be1, Wl0_a, Wr0_a, att0_a, bo0_a, Wl0_r, Wr0_r, att0_r, bo0_r, g2, be2, Wl1_a, Wr1_a, att1_a, bo1_a, Wl1_r, Wr1_r, att1_r, bo1_r, g3, be3, Wz_a, bz_a, Wz_r, bz_r):
    i32 = jnp.int32
    loop = jnp.arange(N, dtype=i32)
    npadE = E_PAD - E_SL
    eiota = jnp.arange(E_PAD, dtype=i32)

    def prep(ei):
        row = jnp.concatenate([ei[0], loop, jnp.zeros((npadE,), i32)])
        col = jnp.concatenate([ei[1], loop, jnp.full((npadE,), NPAD - 1, i32)])
        return row, col

    row_a, col_a = prep(edge_index_atac)
    row_r, col_r = prep(edge_index_rna)

    xp = jnp.pad(x, ((0, NPAD - N), (0, 0)))

    dummy = jnp.zeros((8, HP), jnp.float32)
    deg_a = sc_gsa(eiota, col_a, dummy, d=HP, passes=1, count_mode=True)
    deg_r = sc_gsa(eiota, col_r, dummy, d=HP, passes=1, count_mode=True)

    def gcn(src, W, row, col, deg):
        y = tc_mm_dis(src, W, deg)
        d = W.shape[1]
        return sc_gsa(row, col, y, d=d, passes=4 if d > 256 else 2)

    agg_a = gcn(xp, W1_a, row_a, col_a, deg_a)
    agg_r = gcn(xp, W1_r, row_r, col_r, deg_r)
    h = tc_comb_gcn(agg_a, deg_a, agg_r, deg_r, (b1_a + b1_r).reshape(1, -1))
    h = tc_normsilu(h, tc_stats(h), g1.reshape(1, -1), be1.reshape(1, -1))

    for (Wl_a, Wr_a, att_a, bo_a, Wl_r, Wr_r, att_r, bo_r, g, be) in (
        (Wl0_a, Wr0_a, att0_a, bo0_a, Wl0_r, Wr0_r, att0_r, bo0_r, g2, be2),
        (Wl1_a, Wr1_a, att1_a, bo1_a, Wl1_r, Wr1_r, att1_r, bo1_r, g3, be3),
    ):
        ga = _gat(h, row_a, col_a, eiota, Wl_a, Wr_a, att_a)
        gr = _gat(h, row_r, col_r, eiota, Wl_r, Wr_r, att_r)
        h = tc_comb_add(ga, gr, (bo_a + bo_r).reshape(1, -1))
        h = tc_normsilu(h, tc_stats(h), g.reshape(1, -1), be.reshape(1, -1))

    za = gcn(h, Wz_a, row_a, col_a, deg_a)
    zr = gcn(h, Wz_r, row_r, col_r, deg_r)
    z = tc_comb_gcn(za, deg_a, zr, deg_r, (bz_a + bz_r).reshape(1, -1))
    return z[:N]
